# baseline (speedup 1.0000x reference)
import functools

import jax
import jax.numpy as jnp
from jax import lax
from jax.experimental import pallas as pl
from jax.experimental.pallas import tpu as pltpu
from jax.experimental.pallas import tpu_sc as plsc

EMBED_DIM = 64
HIDDEN_DIM = 256
BATCH = 16384

NUM_CORES = 2
NUM_SUBCORES = 16
NUM_WORKERS = NUM_CORES * NUM_SUBCORES
B_PER_W = BATCH // NUM_WORKERS
CHUNK = 128
CHUNKS_PER_W = B_PER_W // CHUNK


def _sc_gather_body(tidx_hbm, cidx_hbm, ttab_hbm, ctab_hbm,
                    tout_hbm, cout_hbm,
                    tidx_v, cidx_v, trows_v, crows_v, sem):
    wid = lax.axis_index("s") * NUM_CORES + lax.axis_index("c")
    base = wid * B_PER_W
    pltpu.sync_copy(tidx_hbm.at[pl.ds(wid * CHUNKS_PER_W, CHUNKS_PER_W)], tidx_v)
    pltpu.sync_copy(cidx_hbm.at[pl.ds(wid * CHUNKS_PER_W, CHUNKS_PER_W)], cidx_v)
    copies = []
    for j in range(CHUNKS_PER_W):
        copies.append(pltpu.async_copy(
            ttab_hbm.at[tidx_v.at[j]], trows_v.at[pl.ds(j * CHUNK, CHUNK)], sem))
    for j in range(CHUNKS_PER_W):
        copies.append(pltpu.async_copy(
            ctab_hbm.at[cidx_v.at[j]], crows_v.at[pl.ds(j * CHUNK, CHUNK)], sem))
    for c in copies:
        c.wait()
    pltpu.sync_copy(trows_v, tout_hbm.at[pl.ds(base, B_PER_W)])
    pltpu.sync_copy(crows_v, cout_hbm.at[pl.ds(base, B_PER_W)])


_sc_gather = functools.partial(
    pl.kernel,
    mesh=plsc.VectorSubcoreMesh(
        core_axis_name="c", subcore_axis_name="s",
        num_cores=NUM_CORES, num_subcores=NUM_SUBCORES),
    out_type=(
        jax.ShapeDtypeStruct((BATCH, EMBED_DIM), jnp.float32),
        jax.ShapeDtypeStruct((BATCH, EMBED_DIM), jnp.float32),
    ),
    scratch_types=[
        pltpu.VMEM((CHUNKS_PER_W, CHUNK), jnp.int32),
        pltpu.VMEM((CHUNKS_PER_W, CHUNK), jnp.int32),
        pltpu.VMEM((B_PER_W, EMBED_DIM), jnp.float32),
        pltpu.VMEM((B_PER_W, EMBED_DIM), jnp.float32),
        pltpu.SemaphoreType.DMA,
    ],
    compiler_params=pltpu.CompilerParams(use_tc_tiling_on_sc=False),
)(_sc_gather_body)


def _mlp_body(x1_ref, x2_ref, w1a_ref, w1b_ref, b1_ref, w2_ref, b2_ref, o_ref):
    h = jnp.dot(x1_ref[...], w1a_ref[...], preferred_element_type=jnp.float32)
    h = h + jnp.dot(x2_ref[...], w1b_ref[...], preferred_element_type=jnp.float32)
    h = jnp.maximum(h + b1_ref[...], 0.0)
    out = jnp.dot(h, w2_ref[...], preferred_element_type=jnp.float32) + b2_ref[...]
    norm = jnp.sqrt(jnp.sum(out * out, axis=1, keepdims=True))
    o_ref[...] = out / jnp.maximum(norm, 1e-12)


def _mlp(trows, crows, w1a, w1b, b1, w2, b2, block_m=2048):
    grid = (BATCH // block_m,)
    return pl.pallas_call(
        _mlp_body,
        grid=grid,
        in_specs=[
            pl.BlockSpec((block_m, EMBED_DIM), lambda i: (i, 0)),
            pl.BlockSpec((block_m, EMBED_DIM), lambda i: (i, 0)),
            pl.BlockSpec((EMBED_DIM, HIDDEN_DIM), lambda i: (0, 0)),
            pl.BlockSpec((EMBED_DIM, HIDDEN_DIM), lambda i: (0, 0)),
            pl.BlockSpec((1, HIDDEN_DIM), lambda i: (0, 0)),
            pl.BlockSpec((HIDDEN_DIM, EMBED_DIM), lambda i: (0, 0)),
            pl.BlockSpec((1, EMBED_DIM), lambda i: (0, 0)),
        ],
        out_specs=pl.BlockSpec((block_m, EMBED_DIM), lambda i: (i, 0)),
        out_shape=jax.ShapeDtypeStruct((BATCH, EMBED_DIM), jnp.float32),
    )(trows, crows, w1a, w1b, b1, w2, b2)


def kernel(title_idx, category_idx, title_table, category_table, W1, b1, W2, b2):
    tidx2 = title_idx.astype(jnp.int32).reshape(BATCH // CHUNK, CHUNK)
    cidx2 = category_idx.astype(jnp.int32).reshape(BATCH // CHUNK, CHUNK)
    trows, crows = _sc_gather(tidx2, cidx2, title_table, category_table)
    w1t = W1.T
    w1a = w1t[:EMBED_DIM]
    w1b = w1t[EMBED_DIM:]
    return _mlp(trows, crows, w1a, w1b, b1.reshape(1, HIDDEN_DIM),
                W2.T, b2.reshape(1, EMBED_DIM))
